# baseline (speedup 1.0000x reference)
import jax
import jax.numpy as jnp
from jax import lax
from jax.experimental import pallas as pl
from jax.experimental.pallas import tpu as pltpu
from jax.experimental.pallas import tpu_sc as plsc

_MAX_BEAT = 256
_MAX_DUR = 192
_L = 16
_NC = 2
_NS = 16
_NW = _NC * _NS
_F = 129
_B, _S = 256, 2048
_T = _B * _S
_CT = 8192
_NCHUNK = _T // _CT
_NROW = 603

_OFF_POS = 129
_OFF_BEAT = 129 + 24
_OFF_DUR = 129 + 24 + 257


def _mask_body(p_ref, o_ref):
    o_ref[...] = p_ref[...] != 0


def _sc_body(tallT_hbm, freq_hbm, time_hbm, pitch_hbm, dur_hbm, hint_hbm,
             out_hbm,
             tallT_v, freq_v, hint_v,
             ib0, ib1, sa0, sa1, sb0, sb1, sf0, sf1,
             sin0, sin1, sout0, sout1):
    wid = lax.axis_index("s") * _NC + lax.axis_index("c")
    kk = wid >> 3
    j0 = 2 * (wid & 7)
    f0 = 2 * wid
    iota = lax.broadcasted_iota(jnp.int32, (_L,), 0)

    pltpu.sync_copy(tallT_hbm, tallT_v)
    pltpu.sync_copy(freq_hbm, freq_v)
    pltpu.sync_copy(hint_hbm, hint_v)

    def issue_in(c, ib, sem):
        src = pl.ds(c * _CT, _CT)

        @pl.when(kk == 0)
        def _():
            pltpu.async_copy(pitch_hbm.at[src], ib, sem)

        @pl.when(jnp.logical_or(kk == 1, kk == 2))
        def _():
            pltpu.async_copy(time_hbm.at[src], ib, sem)

        @pl.when(kk == 3)
        def _():
            pltpu.async_copy(dur_hbm.at[src], ib, sem)

    def wait_in(ib, sem):
        pltpu.make_async_copy(time_hbm.at[pl.ds(0, _CT)], ib, sem).wait()

    def tok_fill(ib, sa, sb, sf, fwid):
        base = j0 * _NROW

        def g_pitch(gi, carry):
            pv = ib[pl.ds(gi * _L, _L)]
            adr = pv + base
            sa[pl.ds(gi * _L, _L)] = plsc.load_gather(tallT_v, [adr])
            sb[pl.ds(gi * _L, _L)] = plsc.load_gather(tallT_v, [adr + _NROW])
            return carry

        def g_pos(gi, carry):
            tv = ib[pl.ds(gi * _L, _L)]
            qv = (tv * 43691) >> 20
            adr = (tv - qv * 24) + (base + _OFF_POS)
            sa[pl.ds(gi * _L, _L)] = plsc.load_gather(tallT_v, [adr])
            sb[pl.ds(gi * _L, _L)] = plsc.load_gather(tallT_v, [adr + _NROW])
            return carry

        def g_beat(gi, carry):
            tv = ib[pl.ds(gi * _L, _L)]
            qv = (tv * 43691) >> 20
            adr = jnp.minimum(qv, _MAX_BEAT) + (base + _OFF_BEAT)
            sa[pl.ds(gi * _L, _L)] = plsc.load_gather(tallT_v, [adr])
            sb[pl.ds(gi * _L, _L)] = plsc.load_gather(tallT_v, [adr + _NROW])
            return carry

        def g_dur(gi, carry):
            dv = ib[pl.ds(gi * _L, _L)]
            adr = jnp.minimum(dv, _MAX_DUR) + (base + _OFF_DUR)
            sa[pl.ds(gi * _L, _L)] = plsc.load_gather(tallT_v, [adr])
            sb[pl.ds(gi * _L, _L)] = plsc.load_gather(tallT_v, [adr + _NROW])
            return carry

        def g_freq(gi, carry):
            pv = ib[pl.ds(gi * _L, _L)]
            sf[pl.ds(gi * _L, _L)] = plsc.load_gather(freq_v, [pv])
            return carry

        @pl.when(kk == 0)
        def _():
            lax.fori_loop(0, _CT // _L, g_pitch, 0, unroll=2)

        @pl.when(kk == 1)
        def _():
            lax.fori_loop(0, _CT // _L, g_pos, 0, unroll=2)

        @pl.when(kk == 2)
        def _():
            lax.fori_loop(0, _CT // _L, g_beat, 0, unroll=2)

        @pl.when(kk == 3)
        def _():
            lax.fori_loop(0, _CT // _L, g_dur, 0, unroll=2)

        @pl.when(wid == fwid)
        def _():
            lax.fori_loop(0, _CT // _L, g_freq, 0, unroll=2)

    def issue_out(c, sa, sb, sf, fwid, sem):
        dst = pl.ds(f0 * _T + c * _CT, _CT)
        pltpu.async_copy(sa, out_hbm.at[dst], sem)
        dstb = pl.ds((f0 + 1) * _T + c * _CT, _CT)
        pltpu.async_copy(sb, out_hbm.at[dstb], sem)

        @pl.when(wid == fwid)
        def _():
            pltpu.async_copy(sf, out_hbm.at[pl.ds(64 * _T + c * _CT, _CT)],
                             sem)

    def wait_out(sa, sb, sf, fwid, sem):
        pltpu.make_async_copy(sa, out_hbm.at[pl.ds(0, _CT)], sem).wait()
        pltpu.make_async_copy(sb, out_hbm.at[pl.ds(0, _CT)], sem).wait()

        @pl.when(wid == fwid)
        def _():
            pltpu.make_async_copy(sf, out_hbm.at[pl.ds(0, _CT)], sem).wait()

    issue_in(0, ib0, sin0)
    issue_in(1, ib1, sin1)

    def tok_pair(m, carry):
        c0 = 2 * m
        c1 = 2 * m + 1

        wait_in(ib0, sin0)

        @pl.when(m >= 1)
        def _():
            wait_out(sa0, sb0, sf0, 0, sout0)

        tok_fill(ib0, sa0, sb0, sf0, 0)
        issue_out(c0, sa0, sb0, sf0, 0, sout0)

        @pl.when(m < _NCHUNK // 2 - 1)
        def _():
            issue_in(c0 + 2, ib0, sin0)

        wait_in(ib1, sin1)

        @pl.when(m >= 1)
        def _():
            wait_out(sa1, sb1, sf1, 1, sout1)

        tok_fill(ib1, sa1, sb1, sf1, 1)
        issue_out(c1, sa1, sb1, sf1, 1, sout1)

        @pl.when(m < _NCHUNK // 2 - 1)
        def _():
            issue_in(c1 + 2, ib1, sin1)

        return carry

    lax.fori_loop(0, _NCHUNK // 2, tok_pair, 0)
    wait_out(sa0, sb0, sf0, 0, sout0)
    wait_out(sa1, sb1, sf1, 1, sout1)

    qh = (2 * wid) >> 4
    jh = (2 * wid) & 15
    m0 = 65 + 2 * wid

    def hint_fill(c, sa, sb):
        bt = c // 2
        bvec = (bt * 8 + jnp.minimum(iota, 7)) * 4 + qh
        phv = plsc.load_gather(hint_v, [bvec])
        va = plsc.load_gather(tallT_v, [jh * _NROW + phv])
        vb = plsc.load_gather(tallT_v, [(jh + 1) * _NROW + phv])
        for bi in range(8):
            sva = jnp.full((_L,), va[bi], jnp.float32)
            svb = jnp.full((_L,), vb[bi], jnp.float32)

            def w_body(st, carry):
                off = st * 1024 + bi * 128
                for jj in range(8):
                    sa[pl.ds(off + jj * _L, _L)] = sva
                    sb[pl.ds(off + jj * _L, _L)] = svb
                return carry

            lax.fori_loop(0, 8, w_body, 0)

    def issue_hout(c, sa, sb, sem):
        pltpu.async_copy(sa, out_hbm.at[pl.ds(m0 * _T + c * _CT, _CT)], sem)
        pltpu.async_copy(sb, out_hbm.at[pl.ds((m0 + 1) * _T + c * _CT, _CT)],
                         sem)

    def wait_hout(sa, sb, sem):
        pltpu.make_async_copy(sa, out_hbm.at[pl.ds(0, _CT)], sem).wait()
        pltpu.make_async_copy(sb, out_hbm.at[pl.ds(0, _CT)], sem).wait()

    def hint_pair(m, carry):
        c0 = 2 * m
        c1 = 2 * m + 1

        @pl.when(m >= 1)
        def _():
            wait_hout(sa0, sb0, sout0)

        hint_fill(c0, sa0, sb0)
        issue_hout(c0, sa0, sb0, sout0)

        @pl.when(m >= 1)
        def _():
            wait_hout(sa1, sb1, sout1)

        hint_fill(c1, sa1, sb1)
        issue_hout(c1, sa1, sb1, sout1)
        return carry

    lax.fori_loop(0, _NCHUNK // 2, hint_pair, 0)
    wait_hout(sa0, sb0, sout0)
    wait_hout(sa1, sb1, sout1)


def kernel(time, pitch, duration, pitch_hint, W_pitch, W_pos, W_dur, W_beat,
           freq_table):
    B, S = time.shape
    tallT = jnp.concatenate([W_pitch, W_pos, W_beat, W_dur],
                            axis=0).T.reshape(-1)
    freq = jnp.pad(freq_table.reshape(-1), (0, 7))

    mesh = plsc.VectorSubcoreMesh(core_axis_name="c", subcore_axis_name="s",
                                  num_cores=_NC, num_subcores=_NS)
    buf_i = pltpu.VMEM((_CT,), jnp.int32)
    buf_f = pltpu.VMEM((_CT,), jnp.float32)
    sc = pl.kernel(
        _sc_body,
        out_type=jax.ShapeDtypeStruct((_F * _T,), jnp.float32),
        mesh=mesh,
        compiler_params=pltpu.CompilerParams(needs_layout_passes=False),
        scratch_types=[
            pltpu.VMEM((16 * _NROW,), jnp.float32),
            pltpu.VMEM((136,), jnp.float32),
            pltpu.VMEM((B * 4,), jnp.int32),
            buf_i, buf_i, buf_f, buf_f, buf_f, buf_f, buf_f, buf_f,
            pltpu.SemaphoreType.DMA,
            pltpu.SemaphoreType.DMA,
            pltpu.SemaphoreType.DMA,
            pltpu.SemaphoreType.DMA,
        ],
    )
    def tiled_flat(x):
        return x.reshape(B // 8, 8, S // 128, 128).transpose(
            0, 2, 1, 3).reshape(-1)

    out_flat = sc(tallT, freq, tiled_flat(time), tiled_flat(pitch),
                  tiled_flat(duration), pitch_hint.reshape(-1))
    x = out_flat.reshape(_F, _B // 8, _S // 128, 8, 128)
    tensor_out = x.transpose(1, 3, 2, 4, 0).reshape(_B, _S, _F)

    mask = pl.pallas_call(
        _mask_body,
        out_shape=jax.ShapeDtypeStruct((B, S), jnp.bool_),
        grid=(B // 8,),
        in_specs=[pl.BlockSpec((8, S), lambda i: (i, 0))],
        out_specs=pl.BlockSpec((8, S), lambda i: (i, 0)),
    )(pitch)
    return tensor_out, mask


# speedup vs baseline: 1.5375x; 1.5375x over previous
import jax
import jax.numpy as jnp
from jax import lax
from jax.experimental import pallas as pl
from jax.experimental.pallas import tpu as pltpu
from jax.experimental.pallas import tpu_sc as plsc

_MAX_BEAT = 256
_MAX_DUR = 192
_L = 16
_NC = 2
_NS = 16
_NW = _NC * _NS
_F = 129
_B, _S = 256, 2048
_T = _B * _S
_CT = 8192
_NCHUNK = _T // _CT
_NROW = 603

_OFF_POS = 129
_OFF_BEAT = 129 + 24
_OFF_DUR = 129 + 24 + 257


def _mask_body(p_ref, o_ref):
    o_ref[...] = p_ref[...] != 0


def _sc_body(tallT_hbm, freq_hbm, time_hbm, pitch_hbm, dur_hbm, hint_hbm,
             out_hbm,
             tallT_v, freq_v, hint_v,
             ib0, ib1, sa0, sa1, sb0, sb1, sf0, sf1,
             sin0, sin1, sout0, sout1):
    wid = lax.axis_index("s") * _NC + lax.axis_index("c")
    kk = wid >> 3
    j0 = 2 * (wid & 7)
    f0 = 2 * wid
    iota = lax.broadcasted_iota(jnp.int32, (_L,), 0)

    pltpu.sync_copy(tallT_hbm, tallT_v)
    pltpu.sync_copy(freq_hbm, freq_v)
    pltpu.sync_copy(hint_hbm, hint_v)

    def issue_in(c, ib, sem):
        src = pl.ds(c * _CT, _CT)

        @pl.when(kk == 0)
        def _():
            pltpu.async_copy(pitch_hbm.at[src], ib, sem)

        @pl.when(jnp.logical_or(kk == 1, kk == 2))
        def _():
            pltpu.async_copy(time_hbm.at[src], ib, sem)

        @pl.when(kk == 3)
        def _():
            pltpu.async_copy(dur_hbm.at[src], ib, sem)

    def wait_in(ib, sem):
        pltpu.make_async_copy(time_hbm.at[pl.ds(0, _CT)], ib, sem).wait()

    def tok_fill(ib, sa, sb, sf, fwid):
        base = j0 * _NROW

        def g_pitch(gi, carry):
            pv = ib[pl.ds(gi * _L, _L)]
            adr = pv + base
            sa[pl.ds(gi * _L, _L)] = plsc.load_gather(tallT_v, [adr])
            sb[pl.ds(gi * _L, _L)] = plsc.load_gather(tallT_v, [adr + _NROW])
            return carry

        def g_pos(gi, carry):
            tv = ib[pl.ds(gi * _L, _L)]
            qv = (tv * 43691) >> 20
            adr = (tv - qv * 24) + (base + _OFF_POS)
            sa[pl.ds(gi * _L, _L)] = plsc.load_gather(tallT_v, [adr])
            sb[pl.ds(gi * _L, _L)] = plsc.load_gather(tallT_v, [adr + _NROW])
            return carry

        def g_beat(gi, carry):
            tv = ib[pl.ds(gi * _L, _L)]
            qv = (tv * 43691) >> 20
            adr = jnp.minimum(qv, _MAX_BEAT) + (base + _OFF_BEAT)
            sa[pl.ds(gi * _L, _L)] = plsc.load_gather(tallT_v, [adr])
            sb[pl.ds(gi * _L, _L)] = plsc.load_gather(tallT_v, [adr + _NROW])
            return carry

        def g_dur(gi, carry):
            dv = ib[pl.ds(gi * _L, _L)]
            adr = jnp.minimum(dv, _MAX_DUR) + (base + _OFF_DUR)
            sa[pl.ds(gi * _L, _L)] = plsc.load_gather(tallT_v, [adr])
            sb[pl.ds(gi * _L, _L)] = plsc.load_gather(tallT_v, [adr + _NROW])
            return carry

        def g_freq(gi, carry):
            pv = ib[pl.ds(gi * _L, _L)]
            sf[pl.ds(gi * _L, _L)] = plsc.load_gather(freq_v, [pv])
            return carry

        @pl.when(kk == 0)
        def _():
            lax.fori_loop(0, _CT // _L, g_pitch, 0)

        @pl.when(kk == 1)
        def _():
            lax.fori_loop(0, _CT // _L, g_pos, 0)

        @pl.when(kk == 2)
        def _():
            lax.fori_loop(0, _CT // _L, g_beat, 0)

        @pl.when(kk == 3)
        def _():
            lax.fori_loop(0, _CT // _L, g_dur, 0)

        @pl.when(wid == fwid)
        def _():
            lax.fori_loop(0, _CT // _L, g_freq, 0)

    def issue_out(c, sa, sb, sf, fwid, sem):
        dst = pl.ds(f0 * _T + c * _CT, _CT)
        pltpu.async_copy(sa, out_hbm.at[dst], sem)
        dstb = pl.ds((f0 + 1) * _T + c * _CT, _CT)
        pltpu.async_copy(sb, out_hbm.at[dstb], sem)

        @pl.when(wid == fwid)
        def _():
            pltpu.async_copy(sf, out_hbm.at[pl.ds(64 * _T + c * _CT, _CT)],
                             sem)

    def wait_out(sa, sb, sf, fwid, sem):
        pltpu.make_async_copy(sa, out_hbm.at[pl.ds(0, _CT)], sem).wait()
        pltpu.make_async_copy(sb, out_hbm.at[pl.ds(0, _CT)], sem).wait()

        @pl.when(wid == fwid)
        def _():
            pltpu.make_async_copy(sf, out_hbm.at[pl.ds(0, _CT)], sem).wait()

    issue_in(0, ib0, sin0)
    issue_in(1, ib1, sin1)

    def tok_pair(m, carry):
        c0 = 2 * m
        c1 = 2 * m + 1

        wait_in(ib0, sin0)

        @pl.when(m >= 1)
        def _():
            wait_out(sa0, sb0, sf0, 0, sout0)

        tok_fill(ib0, sa0, sb0, sf0, 0)
        issue_out(c0, sa0, sb0, sf0, 0, sout0)

        @pl.when(m < _NCHUNK // 2 - 1)
        def _():
            issue_in(c0 + 2, ib0, sin0)

        wait_in(ib1, sin1)

        @pl.when(m >= 1)
        def _():
            wait_out(sa1, sb1, sf1, 1, sout1)

        tok_fill(ib1, sa1, sb1, sf1, 1)
        issue_out(c1, sa1, sb1, sf1, 1, sout1)

        @pl.when(m < _NCHUNK // 2 - 1)
        def _():
            issue_in(c1 + 2, ib1, sin1)

        return carry

    lax.fori_loop(0, _NCHUNK // 2, tok_pair, 0)
    wait_out(sa0, sb0, sf0, 0, sout0)
    wait_out(sa1, sb1, sf1, 1, sout1)

    qh = (2 * wid) >> 4
    jh = (2 * wid) & 15
    m0 = 65 + 2 * wid

    def hint_fill(c, sa, sb):
        bt = c // 2
        bvec = (bt * 8 + jnp.minimum(iota, 7)) * 4 + qh
        phv = plsc.load_gather(hint_v, [bvec])
        va = plsc.load_gather(tallT_v, [jh * _NROW + phv])
        vb = plsc.load_gather(tallT_v, [(jh + 1) * _NROW + phv])
        for bi in range(8):
            sva = jnp.full((_L,), va[bi], jnp.float32)
            svb = jnp.full((_L,), vb[bi], jnp.float32)

            def w_body(st, carry):
                off = st * 1024 + bi * 128
                for jj in range(8):
                    sa[pl.ds(off + jj * _L, _L)] = sva
                    sb[pl.ds(off + jj * _L, _L)] = svb
                return carry

            lax.fori_loop(0, 8, w_body, 0)

    def issue_hout(c, sa, sb, sem):
        pltpu.async_copy(sa, out_hbm.at[pl.ds(m0 * _T + c * _CT, _CT)], sem)
        pltpu.async_copy(sb, out_hbm.at[pl.ds((m0 + 1) * _T + c * _CT, _CT)],
                         sem)

    def wait_hout(sa, sb, sem):
        pltpu.make_async_copy(sa, out_hbm.at[pl.ds(0, _CT)], sem).wait()
        pltpu.make_async_copy(sb, out_hbm.at[pl.ds(0, _CT)], sem).wait()

    def hint_pair(m, carry):
        c0 = 2 * m
        c1 = 2 * m + 1

        @pl.when(m >= 1)
        def _():
            wait_hout(sa0, sb0, sout0)

        hint_fill(c0, sa0, sb0)
        issue_hout(c0, sa0, sb0, sout0)

        @pl.when(m >= 1)
        def _():
            wait_hout(sa1, sb1, sout1)

        hint_fill(c1, sa1, sb1)
        issue_hout(c1, sa1, sb1, sout1)
        return carry

    lax.fori_loop(0, _NCHUNK // 2, hint_pair, 0)
    wait_hout(sa0, sb0, sout0)
    wait_hout(sa1, sb1, sout1)


def kernel(time, pitch, duration, pitch_hint, W_pitch, W_pos, W_dur, W_beat,
           freq_table):
    B, S = time.shape
    tallT = jnp.concatenate([W_pitch, W_pos, W_beat, W_dur],
                            axis=0).T.reshape(-1)
    freq = jnp.pad(freq_table.reshape(-1), (0, 7))

    mesh = plsc.VectorSubcoreMesh(core_axis_name="c", subcore_axis_name="s",
                                  num_cores=_NC, num_subcores=_NS)
    buf_i = pltpu.VMEM((_CT,), jnp.int32)
    buf_f = pltpu.VMEM((_CT,), jnp.float32)
    sc = pl.kernel(
        _sc_body,
        out_type=jax.ShapeDtypeStruct((_F * _T,), jnp.float32),
        mesh=mesh,
        compiler_params=pltpu.CompilerParams(needs_layout_passes=False),
        scratch_types=[
            pltpu.VMEM((16 * _NROW,), jnp.float32),
            pltpu.VMEM((136,), jnp.float32),
            pltpu.VMEM((B * 4,), jnp.int32),
            buf_i, buf_i, buf_f, buf_f, buf_f, buf_f, buf_f, buf_f,
            pltpu.SemaphoreType.DMA,
            pltpu.SemaphoreType.DMA,
            pltpu.SemaphoreType.DMA,
            pltpu.SemaphoreType.DMA,
        ],
    )
    def tiled_flat(x):
        return x.reshape(B // 8, 8, S // 128, 128).transpose(
            0, 2, 1, 3).reshape(-1)

    out_flat = sc(tallT, freq, tiled_flat(time), tiled_flat(pitch),
                  tiled_flat(duration), pitch_hint.reshape(-1))
    x = out_flat.reshape(_F, _B // 8, _S // 128, 8, 128)
    tensor_out = x.transpose(1, 3, 2, 4, 0).reshape(_B, _S, _F)

    mask = pl.pallas_call(
        _mask_body,
        out_shape=jax.ShapeDtypeStruct((B, S), jnp.bool_),
        grid=(B // 8,),
        in_specs=[pl.BlockSpec((8, S), lambda i: (i, 0))],
        out_specs=pl.BlockSpec((8, S), lambda i: (i, 0)),
    )(pitch)
    return tensor_out, mask
